# baseline (device time: 823278 ns/iter reference)
import jax
import jax.numpy as jnp
from jax import lax
from jax.experimental import pallas as pl
from jax.experimental.pallas import tpu as pltpu

P = 16


def kernel(x, Win0, Wout0, Win1, Wout1, Win2, Wout2):
    B, D = x.shape
    H = Win0.shape[1]

    def body(x_ref, win0, wout0, win1, wout1, win2, wout2,
             act_ref, partial_ref, staging_ref,
             ag_send, ag_recv, rs_send, rs_recv):
        me = lax.axis_index("i")
        left = (me - 1 + P) % P
        right = (me + 1) % P

        barrier_sem = pltpu.get_barrier_semaphore()

        def barrier():
            for nbr in (left, right):
                pl.semaphore_signal(
                    barrier_sem, inc=1,
                    device_id=(nbr,), device_id_type=pl.DeviceIdType.MESH,
                )
            pl.semaphore_wait(barrier_sem, 2)

        def ring_allgather():
            for h in range(P - 1):
                snd = (me - h + 2 * P) % P
                rcv = (me - 1 - h + 2 * P) % P
                send = pltpu.make_async_remote_copy(
                    src_ref=act_ref.at[snd],
                    dst_ref=act_ref.at[snd],
                    send_sem=ag_send.at[snd],
                    recv_sem=ag_recv.at[snd],
                    device_id=(right,),
                    device_id_type=pl.DeviceIdType.MESH,
                )
                send.start()
                recv = pltpu.make_async_remote_copy(
                    src_ref=act_ref.at[rcv],
                    dst_ref=act_ref.at[rcv],
                    send_sem=ag_send.at[rcv],
                    recv_sem=ag_recv.at[rcv],
                    device_id=(right,),
                    device_id_type=pl.DeviceIdType.MESH,
                )
                recv.wait_recv()
                send.wait_send()

        def ring_reducescatter():
            first = (me - 1 + P) % P
            staging_ref[first] = partial_ref[first]
            for s in range(P - 1):
                snd = (me - 1 - s + 2 * P) % P
                rcv = (me - 2 - s + 2 * P) % P
                send = pltpu.make_async_remote_copy(
                    src_ref=staging_ref.at[snd],
                    dst_ref=staging_ref.at[snd],
                    send_sem=rs_send.at[snd],
                    recv_sem=rs_recv.at[snd],
                    device_id=(right,),
                    device_id_type=pl.DeviceIdType.MESH,
                )
                send.start()
                recv = pltpu.make_async_remote_copy(
                    src_ref=staging_ref.at[rcv],
                    dst_ref=staging_ref.at[rcv],
                    send_sem=rs_send.at[rcv],
                    recv_sem=rs_recv.at[rcv],
                    device_id=(right,),
                    device_id_type=pl.DeviceIdType.MESH,
                )
                recv.wait_recv()
                staging_ref[rcv] = staging_ref[rcv] + partial_ref[rcv]
                send.wait_send()

        def compute_layer(win, wout):
            for c in range(P):
                h = jnp.dot(act_ref[c], win[...],
                            preferred_element_type=jnp.float32)
                h = jnp.maximum(h, 0.0)
                partial_ref[c] = jnp.dot(h, wout[...],
                                         preferred_element_type=jnp.float32)

        barrier()
        act_ref[me] = x_ref[...]
        ring_allgather()

        for win, wout in ((win0, wout0), (win1, wout1), (win2, wout2)):
            compute_layer(win, wout)
            barrier()
            ring_reducescatter()
            act_ref[me] = staging_ref[me]
            barrier()
            ring_allgather()

    out = pl.pallas_call(
        body,
        out_shape=jax.ShapeDtypeStruct((P, B, D), jnp.float32),
        in_specs=[pl.BlockSpec(memory_space=pltpu.VMEM)] * 7,
        out_specs=pl.BlockSpec(memory_space=pltpu.VMEM),
        scratch_shapes=[
            pltpu.VMEM((P, B, D), jnp.float32),
            pltpu.VMEM((P, B, D), jnp.float32),
            pltpu.SemaphoreType.DMA((P,)),
            pltpu.SemaphoreType.DMA((P,)),
            pltpu.SemaphoreType.DMA((P,)),
            pltpu.SemaphoreType.DMA((P,)),
        ],
        compiler_params=pltpu.CompilerParams(collective_id=0),
    )(x, Win0, Wout0, Win1, Wout1, Win2, Wout2)
    return out.reshape(P * B, D)


# device time: 530011 ns/iter; 1.5533x vs baseline; 1.5533x over previous
import jax
import jax.numpy as jnp
from jax import lax
from jax.experimental import pallas as pl
from jax.experimental.pallas import tpu as pltpu

P = 16
HCW = 8
HCCW = 7


def kernel(x, Win0, Wout0, Win1, Wout1, Win2, Wout2):
    B, D = x.shape
    H = Win0.shape[1]

    def body(x_ref, win0, wout0, win1, wout1, win2, wout2,
             act_ref, partial_ref, stg_cw, stg_ccw,
             snd_cw, snd_ccw, rcv_cw, rcv_ccw):
        me = lax.axis_index("i")
        left = (me - 1 + P) % P
        right = (me + 1) % P

        barrier_sem = pltpu.get_barrier_semaphore()

        def barrier():
            for nbr in (left, right):
                pl.semaphore_signal(
                    barrier_sem, inc=1,
                    device_id=(nbr,), device_id_type=pl.DeviceIdType.MESH,
                )
            pl.semaphore_wait(barrier_sem, 2)

        def copy(src, dst, ssem, rsem, dev):
            return pltpu.make_async_remote_copy(
                src_ref=src, dst_ref=dst, send_sem=ssem, recv_sem=rsem,
                device_id=(dev,), device_id_type=pl.DeviceIdType.MESH,
            )

        def drain_sends():
            for h in range(HCW):
                copy(act_ref.at[0], act_ref.at[0],
                     snd_cw.at[h], rcv_cw.at[h], right).wait_send()
            for h in range(HCCW):
                copy(act_ref.at[0], act_ref.at[0],
                     snd_ccw.at[h], rcv_ccw.at[h], left).wait_send()

        def ring_allgather(compute_chunk=None):
            for h in range(HCW):
                c_cw = (me - h + 2 * P) % P
                copy(act_ref.at[c_cw], act_ref.at[c_cw],
                     snd_cw.at[h], rcv_cw.at[h], right).start()
                if h < HCCW:
                    c_ccw = (me + h) % P
                    copy(act_ref.at[c_ccw], act_ref.at[c_ccw],
                         snd_ccw.at[h], rcv_ccw.at[h], left).start()
                if h == 0 and compute_chunk is not None:
                    compute_chunk(me)
                a = (me - 1 - h + 2 * P) % P
                copy(act_ref.at[a], act_ref.at[a],
                     snd_cw.at[h], rcv_cw.at[h], right).wait_recv()
                if h < HCCW:
                    b = (me + 1 + h) % P
                    copy(act_ref.at[b], act_ref.at[b],
                         snd_ccw.at[h], rcv_ccw.at[h], left).wait_recv()
                if compute_chunk is not None:
                    compute_chunk(a)
                    if h < HCCW:
                        compute_chunk(b)
            drain_sends()

        def ring_reducescatter():
            for s in range(HCW):
                c_cw = (me + 8 - s) % P
                src = partial_ref.at[c_cw] if s == 0 else stg_cw.at[c_cw]
                copy(src, stg_cw.at[c_cw],
                     snd_cw.at[s], rcv_cw.at[s], right).start()
                if s < HCCW:
                    c_ccw = (me - 7 + s + 2 * P) % P
                    src = partial_ref.at[c_ccw] if s == 0 else stg_ccw.at[c_ccw]
                    copy(src, stg_ccw.at[c_ccw],
                         snd_ccw.at[s], rcv_ccw.at[s], left).start()
                r = (me + 7 - s) % P
                copy(partial_ref.at[r], stg_cw.at[r],
                     snd_cw.at[s], rcv_cw.at[s], right).wait_recv()
                if s < HCW - 1:
                    stg_cw[r] = stg_cw[r] + partial_ref[r]
                if s < HCCW:
                    r2 = (me - 6 + s + 2 * P) % P
                    copy(partial_ref.at[r2], stg_ccw.at[r2],
                         snd_ccw.at[s], rcv_ccw.at[s], left).wait_recv()
                    if s < HCCW - 1:
                        stg_ccw[r2] = stg_ccw[r2] + partial_ref[r2]
            act_ref[me] = stg_cw[me] + stg_ccw[me] + partial_ref[me]
            drain_sends()

        def mk_compute(win, wout):
            def compute_chunk(c):
                h = jnp.dot(act_ref[c], win[...],
                            preferred_element_type=jnp.float32)
                h = jnp.maximum(h, 0.0)
                partial_ref[c] = jnp.dot(h, wout[...],
                                         preferred_element_type=jnp.float32)
            return compute_chunk

        layers = (mk_compute(win0, wout0),
                  mk_compute(win1, wout1),
                  mk_compute(win2, wout2))

        barrier()
        act_ref[me] = x_ref[...]
        ring_allgather(compute_chunk=layers[0])
        for l in range(3):
            barrier()
            ring_reducescatter()
            barrier()
            ring_allgather(
                compute_chunk=layers[l + 1] if l < 2 else None)

    out = pl.pallas_call(
        body,
        out_shape=jax.ShapeDtypeStruct((P, B, D), jnp.float32),
        in_specs=[pl.BlockSpec(memory_space=pltpu.VMEM)] * 7,
        out_specs=pl.BlockSpec(memory_space=pltpu.VMEM),
        scratch_shapes=[
            pltpu.VMEM((P, B, D), jnp.float32),
            pltpu.VMEM((P, B, D), jnp.float32),
            pltpu.VMEM((P, B, D), jnp.float32),
            pltpu.SemaphoreType.DMA((HCW,)),
            pltpu.SemaphoreType.DMA((HCCW,)),
            pltpu.SemaphoreType.DMA((HCW,)),
            pltpu.SemaphoreType.DMA((HCCW,)),
        ],
        compiler_params=pltpu.CompilerParams(collective_id=0),
    )(x, Win0, Wout0, Win1, Wout1, Win2, Wout2)
    return out.reshape(P * B, D)


# device time: 511277 ns/iter; 1.6102x vs baseline; 1.0366x over previous
import jax
import jax.numpy as jnp
from jax import lax
from jax.experimental import pallas as pl
from jax.experimental.pallas import tpu as pltpu

P = 16
HCW = 8
HCCW = 7


def kernel(x, Win0, Wout0, Win1, Wout1, Win2, Wout2):
    B, D = x.shape
    H = Win0.shape[1]

    def body(x_ref, win0, wout0, win1, wout1, win2, wout2,
             act_ref, partial_ref, stg_cw, stg_ccw,
             snd_cw, snd_ccw, rcv_cw, rcv_ccw):
        me = lax.axis_index("i")
        left = (me - 1 + P) % P
        right = (me + 1) % P

        barrier_sem = pltpu.get_barrier_semaphore()

        def barrier():
            for nbr in (left, right):
                pl.semaphore_signal(
                    barrier_sem, inc=1,
                    device_id=(nbr,), device_id_type=pl.DeviceIdType.MESH,
                )
            pl.semaphore_wait(barrier_sem, 2)

        def copy(src, dst, ssem, rsem, dev):
            return pltpu.make_async_remote_copy(
                src_ref=src, dst_ref=dst, send_sem=ssem, recv_sem=rsem,
                device_id=(dev,), device_id_type=pl.DeviceIdType.MESH,
            )

        def drain_sends():
            for h in range(HCW):
                copy(act_ref.at[0], act_ref.at[0],
                     snd_cw.at[h], rcv_cw.at[h], right).wait_send()
            for h in range(HCCW):
                copy(act_ref.at[0], act_ref.at[0],
                     snd_ccw.at[h], rcv_ccw.at[h], left).wait_send()

        def ring_allgather(compute_chunk=None):
            copy(act_ref.at[me], act_ref.at[me],
                 snd_cw.at[0], rcv_cw.at[0], right).start()
            copy(act_ref.at[me], act_ref.at[me],
                 snd_ccw.at[0], rcv_ccw.at[0], left).start()
            if compute_chunk is not None:
                compute_chunk(me)
            for h in range(HCW):
                a = (me - 1 - h + 2 * P) % P
                copy(act_ref.at[a], act_ref.at[a],
                     snd_cw.at[h], rcv_cw.at[h], right).wait_recv()
                b = (me + 1 + h) % P
                if h < HCCW:
                    copy(act_ref.at[b], act_ref.at[b],
                         snd_ccw.at[h], rcv_ccw.at[h], left).wait_recv()
                if h + 1 < HCW:
                    copy(act_ref.at[a], act_ref.at[a],
                         snd_cw.at[h + 1], rcv_cw.at[h + 1], right).start()
                if h + 1 < HCCW:
                    copy(act_ref.at[b], act_ref.at[b],
                         snd_ccw.at[h + 1], rcv_ccw.at[h + 1], left).start()
                if compute_chunk is not None:
                    compute_chunk(a)
                    if h < HCCW:
                        compute_chunk(b)
            drain_sends()

        def ring_reducescatter():
            for s in range(HCW):
                c_cw = (me + 8 - s) % P
                src = partial_ref.at[c_cw] if s == 0 else stg_cw.at[c_cw]
                copy(src, stg_cw.at[c_cw],
                     snd_cw.at[s], rcv_cw.at[s], right).start()
                if s < HCCW:
                    c_ccw = (me - 7 + s + 2 * P) % P
                    src = partial_ref.at[c_ccw] if s == 0 else stg_ccw.at[c_ccw]
                    copy(src, stg_ccw.at[c_ccw],
                         snd_ccw.at[s], rcv_ccw.at[s], left).start()
                r = (me + 7 - s) % P
                copy(partial_ref.at[r], stg_cw.at[r],
                     snd_cw.at[s], rcv_cw.at[s], right).wait_recv()
                if s < HCW - 1:
                    stg_cw[r] = stg_cw[r] + partial_ref[r]
                if s < HCCW:
                    r2 = (me - 6 + s + 2 * P) % P
                    copy(partial_ref.at[r2], stg_ccw.at[r2],
                         snd_ccw.at[s], rcv_ccw.at[s], left).wait_recv()
                    if s < HCCW - 1:
                        stg_ccw[r2] = stg_ccw[r2] + partial_ref[r2]
            act_ref[me] = stg_cw[me] + stg_ccw[me] + partial_ref[me]
            drain_sends()

        def mk_compute(win, wout):
            def compute_chunk(c):
                h = jnp.dot(act_ref[c], win[...],
                            preferred_element_type=jnp.float32)
                h = jnp.maximum(h, 0.0)
                partial_ref[c] = jnp.dot(h, wout[...],
                                         preferred_element_type=jnp.float32)
            return compute_chunk

        layers = (mk_compute(win0, wout0),
                  mk_compute(win1, wout1),
                  mk_compute(win2, wout2))

        barrier()
        act_ref[me] = x_ref[...]
        ring_allgather(compute_chunk=layers[0])
        for l in range(3):
            barrier()
            ring_reducescatter()
            barrier()
            ring_allgather(
                compute_chunk=layers[l + 1] if l < 2 else None)

    out = pl.pallas_call(
        body,
        out_shape=jax.ShapeDtypeStruct((P, B, D), jnp.float32),
        in_specs=[pl.BlockSpec(memory_space=pltpu.VMEM)] * 7,
        out_specs=pl.BlockSpec(memory_space=pltpu.VMEM),
        scratch_shapes=[
            pltpu.VMEM((P, B, D), jnp.float32),
            pltpu.VMEM((P, B, D), jnp.float32),
            pltpu.VMEM((P, B, D), jnp.float32),
            pltpu.SemaphoreType.DMA((HCW,)),
            pltpu.SemaphoreType.DMA((HCCW,)),
            pltpu.SemaphoreType.DMA((HCW,)),
            pltpu.SemaphoreType.DMA((HCCW,)),
        ],
        compiler_params=pltpu.CompilerParams(collective_id=0),
    )(x, Win0, Wout0, Win1, Wout1, Win2, Wout2)
    return out.reshape(P * B, D)


# device time: 509131 ns/iter; 1.6170x vs baseline; 1.0042x over previous
import jax
import jax.numpy as jnp
from jax import lax
from jax.experimental import pallas as pl
from jax.experimental.pallas import tpu as pltpu

P = 16
HCW = 8
HCCW = 7


def kernel(x, Win0, Wout0, Win1, Wout1, Win2, Wout2):
    B, D = x.shape
    H = Win0.shape[1]

    def body(x_ref, win0, wout0, win1, wout1, win2, wout2,
             act_ref, partial_ref, stg_cw, stg_ccw,
             snd_cw, snd_ccw, rcv_cw, rcv_ccw):
        me = lax.axis_index("i")
        left = (me - 1 + P) % P
        right = (me + 1) % P

        barrier_sem = pltpu.get_barrier_semaphore()

        def barrier():
            for nbr in (left, right):
                pl.semaphore_signal(
                    barrier_sem, inc=1,
                    device_id=(nbr,), device_id_type=pl.DeviceIdType.MESH,
                )
            pl.semaphore_wait(barrier_sem, 2)

        def copy(src, dst, ssem, rsem, dev):
            return pltpu.make_async_remote_copy(
                src_ref=src, dst_ref=dst, send_sem=ssem, recv_sem=rsem,
                device_id=(dev,), device_id_type=pl.DeviceIdType.MESH,
            )

        def drain_sends():
            for h in range(HCW):
                copy(act_ref.at[0], act_ref.at[0],
                     snd_cw.at[h], rcv_cw.at[h], right).wait_send()
            for h in range(HCCW):
                copy(act_ref.at[0], act_ref.at[0],
                     snd_ccw.at[h], rcv_ccw.at[h], left).wait_send()

        def ring_allgather(compute_chunk=None):
            copy(act_ref.at[me], act_ref.at[me],
                 snd_cw.at[0], rcv_cw.at[0], right).start()
            copy(act_ref.at[me], act_ref.at[me],
                 snd_ccw.at[0], rcv_ccw.at[0], left).start()
            if compute_chunk is not None:
                compute_chunk(me)
            for h in range(HCW):
                a = (me - 1 - h + 2 * P) % P
                copy(act_ref.at[a], act_ref.at[a],
                     snd_cw.at[h], rcv_cw.at[h], right).wait_recv()
                b = (me + 1 + h) % P
                if h < HCCW:
                    copy(act_ref.at[b], act_ref.at[b],
                         snd_ccw.at[h], rcv_ccw.at[h], left).wait_recv()
                if h + 1 < HCW:
                    copy(act_ref.at[a], act_ref.at[a],
                         snd_cw.at[h + 1], rcv_cw.at[h + 1], right).start()
                if h + 1 < HCCW:
                    copy(act_ref.at[b], act_ref.at[b],
                         snd_ccw.at[h + 1], rcv_ccw.at[h + 1], left).start()
                if compute_chunk is not None:
                    compute_chunk(a)
                    if h < HCCW:
                        compute_chunk(b)
            drain_sends()

        def ring_reducescatter():
            for s in range(HCW):
                c_cw = (me + 8 - s) % P
                src = partial_ref.at[c_cw] if s == 0 else stg_cw.at[c_cw]
                copy(src, stg_cw.at[c_cw],
                     snd_cw.at[s], rcv_cw.at[s], right).start()
                if s < HCCW:
                    c_ccw = (me - 7 + s + 2 * P) % P
                    src = partial_ref.at[c_ccw] if s == 0 else stg_ccw.at[c_ccw]
                    copy(src, stg_ccw.at[c_ccw],
                         snd_ccw.at[s], rcv_ccw.at[s], left).start()
                r = (me + 7 - s) % P
                copy(partial_ref.at[r], stg_cw.at[r],
                     snd_cw.at[s], rcv_cw.at[s], right).wait_recv()
                if s < HCW - 1:
                    stg_cw[r] = stg_cw[r] + partial_ref[r]
                if s < HCCW:
                    r2 = (me - 6 + s + 2 * P) % P
                    copy(partial_ref.at[r2], stg_ccw.at[r2],
                         snd_ccw.at[s], rcv_ccw.at[s], left).wait_recv()
                    if s < HCCW - 1:
                        stg_ccw[r2] = stg_ccw[r2] + partial_ref[r2]
            act_ref[me] = stg_cw[me] + stg_ccw[me] + partial_ref[me]
            drain_sends()

        def mk_compute(win, wout):
            def compute_chunk(c):
                h = jnp.dot(act_ref[c], win[...],
                            preferred_element_type=jnp.float32)
                h = jnp.maximum(h, 0.0)
                partial_ref[c] = jnp.dot(h, wout[...],
                                         preferred_element_type=jnp.float32)
            return compute_chunk

        layers = (mk_compute(win0, wout0),
                  mk_compute(win1, wout1),
                  mk_compute(win2, wout2))

        import os
        _nc = os.environ.get("KERNEL_NO_COMPUTE") == "1"
        barrier()
        act_ref[me] = x_ref[...]
        ring_allgather(compute_chunk=None if _nc else layers[0])
        for l in range(3):
            barrier()
            ring_reducescatter()
            barrier()
            ring_allgather(
                compute_chunk=layers[l + 1] if (l < 2 and not _nc) else None)

    out = pl.pallas_call(
        body,
        out_shape=jax.ShapeDtypeStruct((P, B, D), jnp.float32),
        in_specs=[pl.BlockSpec(memory_space=pltpu.VMEM)] * 7,
        out_specs=pl.BlockSpec(memory_space=pltpu.VMEM),
        scratch_shapes=[
            pltpu.VMEM((P, B, D), jnp.float32),
            pltpu.VMEM((P, B, D), jnp.float32),
            pltpu.VMEM((P, B, D), jnp.float32),
            pltpu.SemaphoreType.DMA((HCW,)),
            pltpu.SemaphoreType.DMA((HCCW,)),
            pltpu.SemaphoreType.DMA((HCW,)),
            pltpu.SemaphoreType.DMA((HCCW,)),
        ],
        compiler_params=pltpu.CompilerParams(collective_id=0),
    )(x, Win0, Wout0, Win1, Wout1, Win2, Wout2)
    return out.reshape(P * B, D)


# device time: 276065 ns/iter; 2.9822x vs baseline; 1.8442x over previous
import jax
import jax.numpy as jnp
from jax import lax
from jax.experimental import pallas as pl
from jax.experimental.pallas import tpu as pltpu

P = 16
HCW = 8
HCCW = 7

ORDER = [0, 4, 8, 12, 13, 9, 5, 1, 2, 6, 10, 14, 15, 11, 7, 3]


def kernel(x, Win0, Wout0, Win1, Wout1, Win2, Wout2):
    B, D = x.shape
    H = Win0.shape[1]

    def body(x_ref, win0, wout0, win1, wout1, win2, wout2,
             out_ref, act_ref, partial_ref, stg_cw, stg_ccw,
             win_bf, wout_bf,
             snd_cw, snd_ccw, rcv_cw, rcv_ccw):
        me = lax.axis_index("i")

        rk = jnp.int32(0)
        for r in range(P):
            rk = jnp.where(me == ORDER[r], jnp.int32(r), rk)

        def ring_at(off):
            v = jnp.int32(0)
            for r in range(P):
                v = jnp.where(rk == r, jnp.int32(ORDER[(r + off) % P]), v)
            return v

        rid = {off: ring_at(off) for off in range(-HCW, HCW + 1)}
        left, right = rid[-1], rid[1]

        barrier_sem = pltpu.get_barrier_semaphore()

        def barrier():
            for nbr in (left, right):
                pl.semaphore_signal(
                    barrier_sem, inc=1,
                    device_id=(nbr,), device_id_type=pl.DeviceIdType.MESH,
                )
            pl.semaphore_wait(barrier_sem, 2)

        def copy(src, dst, ssem, rsem, dev):
            return pltpu.make_async_remote_copy(
                src_ref=src, dst_ref=dst, send_sem=ssem, recv_sem=rsem,
                device_id=(dev,), device_id_type=pl.DeviceIdType.MESH,
            )

        def drain_sends():
            for h in range(HCW):
                copy(act_ref.at[0], act_ref.at[0],
                     snd_cw.at[h], rcv_cw.at[h], right).wait_send()
            for h in range(HCCW):
                copy(act_ref.at[0], act_ref.at[0],
                     snd_ccw.at[h], rcv_ccw.at[h], left).wait_send()

        def ring_allgather(compute_chunk=None):
            copy(act_ref.at[me], act_ref.at[me],
                 snd_cw.at[0], rcv_cw.at[0], right).start()
            copy(act_ref.at[me], act_ref.at[me],
                 snd_ccw.at[0], rcv_ccw.at[0], left).start()
            if compute_chunk is not None:
                compute_chunk(me)
            for h in range(HCW):
                a = rid[-1 - h]
                copy(act_ref.at[a], act_ref.at[a],
                     snd_cw.at[h], rcv_cw.at[h], right).wait_recv()
                b = rid[1 + h]
                if h < HCCW:
                    copy(act_ref.at[b], act_ref.at[b],
                         snd_ccw.at[h], rcv_ccw.at[h], left).wait_recv()
                if h + 1 < HCW:
                    copy(act_ref.at[a], act_ref.at[a],
                         snd_cw.at[h + 1], rcv_cw.at[h + 1], right).start()
                if h + 1 < HCCW:
                    copy(act_ref.at[b], act_ref.at[b],
                         snd_ccw.at[h + 1], rcv_ccw.at[h + 1], left).start()
                if compute_chunk is not None:
                    compute_chunk(a)
                    if h < HCCW:
                        compute_chunk(b)
            drain_sends()

        def ring_reducescatter():
            for s in range(HCW):
                c_cw = rid[8 - s]
                if s == 0:
                    stg_cw[c_cw] = partial_ref[c_cw].astype(jnp.bfloat16)
                copy(stg_cw.at[c_cw], stg_cw.at[c_cw],
                     snd_cw.at[s], rcv_cw.at[s], right).start()
                if s < HCCW:
                    c_ccw = rid[-7 + s]
                    if s == 0:
                        stg_ccw[c_ccw] = partial_ref[c_ccw].astype(
                            jnp.bfloat16)
                    copy(stg_ccw.at[c_ccw], stg_ccw.at[c_ccw],
                         snd_ccw.at[s], rcv_ccw.at[s], left).start()
                r = rid[7 - s]
                copy(stg_cw.at[r], stg_cw.at[r],
                     snd_cw.at[s], rcv_cw.at[s], right).wait_recv()
                if s < HCW - 1:
                    stg_cw[r] = (stg_cw[r] + partial_ref[r]).astype(
                        jnp.bfloat16)
                if s < HCCW:
                    r2 = rid[-6 + s]
                    copy(stg_ccw.at[r2], stg_ccw.at[r2],
                         snd_ccw.at[s], rcv_ccw.at[s], left).wait_recv()
                    if s < HCCW - 1:
                        stg_ccw[r2] = (stg_ccw[r2] + partial_ref[r2]).astype(
                            jnp.bfloat16)
            act_ref[me] = (stg_cw[me] + stg_ccw[me]
                           + partial_ref[me]).astype(jnp.bfloat16)
            drain_sends()

        def mk_compute(l):
            def compute_chunk(c):
                h = jnp.dot(act_ref[c], win_bf[l],
                            preferred_element_type=jnp.float32)
                h = jnp.maximum(h, 0.0)
                partial_ref[c] = jnp.dot(h.astype(jnp.bfloat16), wout_bf[l],
                                         preferred_element_type=jnp.float32)
            return compute_chunk

        for l, (wi, wo) in enumerate(((win0, wout0), (win1, wout1),
                                      (win2, wout2))):
            win_bf[l] = wi[...].astype(jnp.bfloat16)
            wout_bf[l] = wo[...].astype(jnp.bfloat16)

        barrier()
        act_ref[me] = x_ref[...].astype(jnp.bfloat16)
        ring_allgather(compute_chunk=mk_compute(0))
        for l in range(3):
            barrier()
            ring_reducescatter()
            barrier()
            ring_allgather(compute_chunk=mk_compute(l + 1) if l < 2 else None)
        out_ref[...] = act_ref[...].astype(jnp.float32)

    out = pl.pallas_call(
        body,
        out_shape=jax.ShapeDtypeStruct((P, B, D), jnp.float32),
        in_specs=[pl.BlockSpec(memory_space=pltpu.VMEM)] * 7,
        out_specs=pl.BlockSpec(memory_space=pltpu.VMEM),
        scratch_shapes=[
            pltpu.VMEM((P, B, D), jnp.bfloat16),
            pltpu.VMEM((P, B, D), jnp.float32),
            pltpu.VMEM((P, B, D), jnp.bfloat16),
            pltpu.VMEM((P, B, D), jnp.bfloat16),
            pltpu.VMEM((3, D, H), jnp.bfloat16),
            pltpu.VMEM((3, H, D), jnp.bfloat16),
            pltpu.SemaphoreType.DMA((HCW,)),
            pltpu.SemaphoreType.DMA((HCCW,)),
            pltpu.SemaphoreType.DMA((HCW,)),
            pltpu.SemaphoreType.DMA((HCCW,)),
        ],
        compiler_params=pltpu.CompilerParams(collective_id=0),
    )(x, Win0, Wout0, Win1, Wout1, Win2, Wout2)
    return out.reshape(P * B, D)


# device time: 205890 ns/iter; 3.9986x vs baseline; 1.3408x over previous
import jax
import jax.numpy as jnp
from jax import lax
from jax.experimental import pallas as pl
from jax.experimental.pallas import tpu as pltpu

P = 16
HCW = 8
HCCW = 7
SUB = 2

ORDER = [0, 4, 8, 12, 13, 9, 5, 1, 2, 6, 10, 14, 15, 11, 7, 3]


def kernel(x, Win0, Wout0, Win1, Wout1, Win2, Wout2):
    B, D = x.shape
    H = Win0.shape[1]
    Bs = B // SUB

    def body(x_ref, win0, wout0, win1, wout1, win2, wout2,
             out_ref, act_ref, partial_ref, stg_cw, stg_ccw,
             win_bf, wout_bf,
             snd_cw, snd_ccw, rcv_cw, rcv_ccw):
        me = lax.axis_index("i")

        rk = jnp.int32(0)
        for r in range(P):
            rk = jnp.where(me == ORDER[r], jnp.int32(r), rk)

        def ring_at(off):
            v = jnp.int32(0)
            for r in range(P):
                v = jnp.where(rk == r, jnp.int32(ORDER[(r + off) % P]), v)
            return v

        rid = {off: ring_at(off) for off in range(-HCW, HCW + 1)}
        left, right = rid[-1], rid[1]

        barrier_sem = pltpu.get_barrier_semaphore()

        def barrier():
            for nbr in (left, right):
                pl.semaphore_signal(
                    barrier_sem, inc=1,
                    device_id=(nbr,), device_id_type=pl.DeviceIdType.MESH,
                )
            pl.semaphore_wait(barrier_sem, 2)

        def copy(src, dst, ssem, rsem, dev):
            return pltpu.make_async_remote_copy(
                src_ref=src, dst_ref=dst, send_sem=ssem, recv_sem=rsem,
                device_id=(dev,), device_id_type=pl.DeviceIdType.MESH,
            )

        def drain_sends():
            for h in range(HCW):
                for k in range(SUB):
                    copy(act_ref.at[0, k], act_ref.at[0, k],
                         snd_cw.at[h, k], rcv_cw.at[h, k], right).wait_send()
            for h in range(HCCW):
                for k in range(SUB):
                    copy(act_ref.at[0, k], act_ref.at[0, k],
                         snd_ccw.at[h, k], rcv_ccw.at[h, k], left).wait_send()

        def ring_allgather(compute_chunk=None):
            for k in range(SUB):
                copy(act_ref.at[me, k], act_ref.at[me, k],
                     snd_cw.at[0, k], rcv_cw.at[0, k], right).start()
                copy(act_ref.at[me, k], act_ref.at[me, k],
                     snd_ccw.at[0, k], rcv_ccw.at[0, k], left).start()
            if compute_chunk is not None:
                compute_chunk(me)
            for h in range(HCW):
                a = rid[-1 - h]
                b = rid[1 + h]
                for k in range(SUB):
                    copy(act_ref.at[a, k], act_ref.at[a, k],
                         snd_cw.at[h, k], rcv_cw.at[h, k], right).wait_recv()
                    if h + 1 < HCW:
                        copy(act_ref.at[a, k], act_ref.at[a, k],
                             snd_cw.at[h + 1, k], rcv_cw.at[h + 1, k],
                             right).start()
                    if h < HCCW:
                        copy(act_ref.at[b, k], act_ref.at[b, k],
                             snd_ccw.at[h, k], rcv_ccw.at[h, k],
                             left).wait_recv()
                        if h + 1 < HCCW:
                            copy(act_ref.at[b, k], act_ref.at[b, k],
                                 snd_ccw.at[h + 1, k], rcv_ccw.at[h + 1, k],
                                 left).start()
                if compute_chunk is not None:
                    compute_chunk(a)
                    if h < HCCW:
                        compute_chunk(b)
            drain_sends()

        def ring_reducescatter():
            c0, c0c = rid[8], rid[-7]
            for k in range(SUB):
                stg_cw[c0, k] = partial_ref[c0, k].astype(jnp.bfloat16)
                copy(stg_cw.at[c0, k], stg_cw.at[c0, k],
                     snd_cw.at[0, k], rcv_cw.at[0, k], right).start()
                stg_ccw[c0c, k] = partial_ref[c0c, k].astype(jnp.bfloat16)
                copy(stg_ccw.at[c0c, k], stg_ccw.at[c0c, k],
                     snd_ccw.at[0, k], rcv_ccw.at[0, k], left).start()
            for s in range(HCW):
                r = rid[7 - s]
                for k in range(SUB):
                    copy(stg_cw.at[r, k], stg_cw.at[r, k],
                         snd_cw.at[s, k], rcv_cw.at[s, k], right).wait_recv()
                    if s < HCW - 1:
                        stg_cw[r, k] = (stg_cw[r, k]
                                        + partial_ref[r, k]).astype(
                                            jnp.bfloat16)
                        copy(stg_cw.at[r, k], stg_cw.at[r, k],
                             snd_cw.at[s + 1, k], rcv_cw.at[s + 1, k],
                             right).start()
                if s < HCCW:
                    r2 = rid[-6 + s]
                    for k in range(SUB):
                        copy(stg_ccw.at[r2, k], stg_ccw.at[r2, k],
                             snd_ccw.at[s, k], rcv_ccw.at[s, k],
                             left).wait_recv()
                        if s < HCCW - 1:
                            stg_ccw[r2, k] = (stg_ccw[r2, k]
                                              + partial_ref[r2, k]).astype(
                                                  jnp.bfloat16)
                            copy(stg_ccw.at[r2, k], stg_ccw.at[r2, k],
                                 snd_ccw.at[s + 1, k], rcv_ccw.at[s + 1, k],
                                 left).start()
            for k in range(SUB):
                act_ref[me, k] = (stg_cw[me, k] + stg_ccw[me, k]
                                  + partial_ref[me, k]).astype(jnp.bfloat16)
            drain_sends()

        def mk_compute(l):
            def compute_chunk(c):
                for k in range(SUB):
                    h = jnp.dot(act_ref[c, k], win_bf[l],
                                preferred_element_type=jnp.float32)
                    h = jnp.maximum(h, 0.0)
                    partial_ref[c, k] = jnp.dot(
                        h.astype(jnp.bfloat16), wout_bf[l],
                        preferred_element_type=jnp.float32)
            return compute_chunk

        for l, (wi, wo) in enumerate(((win0, wout0), (win1, wout1),
                                      (win2, wout2))):
            win_bf[l] = wi[...].astype(jnp.bfloat16)
            wout_bf[l] = wo[...].astype(jnp.bfloat16)

        barrier()
        for k in range(SUB):
            act_ref[me, k] = x_ref[pl.ds(k * Bs, Bs), :].astype(jnp.bfloat16)
        ring_allgather(compute_chunk=mk_compute(0))
        for l in range(3):
            barrier()
            ring_reducescatter()
            barrier()
            ring_allgather(compute_chunk=mk_compute(l + 1) if l < 2 else None)
        out_ref[...] = act_ref[...].astype(jnp.float32)

    out = pl.pallas_call(
        body,
        out_shape=jax.ShapeDtypeStruct((P, SUB, Bs, D), jnp.float32),
        in_specs=[pl.BlockSpec(memory_space=pltpu.VMEM)] * 7,
        out_specs=pl.BlockSpec(memory_space=pltpu.VMEM),
        scratch_shapes=[
            pltpu.VMEM((P, SUB, Bs, D), jnp.bfloat16),
            pltpu.VMEM((P, SUB, Bs, D), jnp.float32),
            pltpu.VMEM((P, SUB, Bs, D), jnp.bfloat16),
            pltpu.VMEM((P, SUB, Bs, D), jnp.bfloat16),
            pltpu.VMEM((3, D, H), jnp.bfloat16),
            pltpu.VMEM((3, H, D), jnp.bfloat16),
            pltpu.SemaphoreType.DMA((HCW, SUB)),
            pltpu.SemaphoreType.DMA((HCCW, SUB)),
            pltpu.SemaphoreType.DMA((HCW, SUB)),
            pltpu.SemaphoreType.DMA((HCCW, SUB)),
        ],
        compiler_params=pltpu.CompilerParams(collective_id=0),
    )(x, Win0, Wout0, Win1, Wout1, Win2, Wout2)
    return out.reshape(P * B, D)


# device time: 197068 ns/iter; 4.1776x vs baseline; 1.0448x over previous
import jax
import jax.numpy as jnp
from jax import lax
from jax.experimental import pallas as pl
from jax.experimental.pallas import tpu as pltpu

P = 16
HCW = 8
HCCW = 7
SUB = 4

ORDER = [0, 4, 8, 12, 13, 9, 5, 1, 2, 6, 10, 14, 15, 11, 7, 3]


def kernel(x, Win0, Wout0, Win1, Wout1, Win2, Wout2):
    B, D = x.shape
    H = Win0.shape[1]
    Bs = B // SUB

    def body(x_ref, win0, wout0, win1, wout1, win2, wout2,
             out_ref, act_ref, partial_ref, stg_cw, stg_ccw,
             win_bf, wout_bf,
             snd_cw, snd_ccw, rcv_cw, rcv_ccw):
        me = lax.axis_index("i")

        rk = jnp.int32(0)
        for r in range(P):
            rk = jnp.where(me == ORDER[r], jnp.int32(r), rk)

        def ring_at(off):
            v = jnp.int32(0)
            for r in range(P):
                v = jnp.where(rk == r, jnp.int32(ORDER[(r + off) % P]), v)
            return v

        rid = {off: ring_at(off) for off in range(-HCW, HCW + 1)}
        left, right = rid[-1], rid[1]

        barrier_sem = pltpu.get_barrier_semaphore()

        def barrier():
            for nbr in (left, right):
                pl.semaphore_signal(
                    barrier_sem, inc=1,
                    device_id=(nbr,), device_id_type=pl.DeviceIdType.MESH,
                )
            pl.semaphore_wait(barrier_sem, 2)

        def copy(src, dst, ssem, rsem, dev):
            return pltpu.make_async_remote_copy(
                src_ref=src, dst_ref=dst, send_sem=ssem, recv_sem=rsem,
                device_id=(dev,), device_id_type=pl.DeviceIdType.MESH,
            )

        def drain_sends():
            for h in range(HCW):
                for k in range(SUB):
                    copy(act_ref.at[0, k], act_ref.at[0, k],
                         snd_cw.at[h, k], rcv_cw.at[h, k], right).wait_send()
            for h in range(HCCW):
                for k in range(SUB):
                    copy(act_ref.at[0, k], act_ref.at[0, k],
                         snd_ccw.at[h, k], rcv_ccw.at[h, k], left).wait_send()

        def ring_allgather(compute_chunk=None):
            for k in range(SUB):
                copy(act_ref.at[me, k], act_ref.at[me, k],
                     snd_cw.at[0, k], rcv_cw.at[0, k], right).start()
                copy(act_ref.at[me, k], act_ref.at[me, k],
                     snd_ccw.at[0, k], rcv_ccw.at[0, k], left).start()
            if compute_chunk is not None:
                compute_chunk(me)
            for h in range(HCW):
                a = rid[-1 - h]
                b = rid[1 + h]
                for k in range(SUB):
                    copy(act_ref.at[a, k], act_ref.at[a, k],
                         snd_cw.at[h, k], rcv_cw.at[h, k], right).wait_recv()
                    if h + 1 < HCW:
                        copy(act_ref.at[a, k], act_ref.at[a, k],
                             snd_cw.at[h + 1, k], rcv_cw.at[h + 1, k],
                             right).start()
                    if h < HCCW:
                        copy(act_ref.at[b, k], act_ref.at[b, k],
                             snd_ccw.at[h, k], rcv_ccw.at[h, k],
                             left).wait_recv()
                        if h + 1 < HCCW:
                            copy(act_ref.at[b, k], act_ref.at[b, k],
                                 snd_ccw.at[h + 1, k], rcv_ccw.at[h + 1, k],
                                 left).start()
                if compute_chunk is not None:
                    compute_chunk(a)
                    if h < HCCW:
                        compute_chunk(b)
            drain_sends()

        def ring_reducescatter():
            c0, c0c = rid[8], rid[-7]
            for k in range(SUB):
                stg_cw[c0, k] = partial_ref[c0, k].astype(jnp.bfloat16)
                copy(stg_cw.at[c0, k], stg_cw.at[c0, k],
                     snd_cw.at[0, k], rcv_cw.at[0, k], right).start()
                stg_ccw[c0c, k] = partial_ref[c0c, k].astype(jnp.bfloat16)
                copy(stg_ccw.at[c0c, k], stg_ccw.at[c0c, k],
                     snd_ccw.at[0, k], rcv_ccw.at[0, k], left).start()
            for s in range(HCW):
                r = rid[7 - s]
                for k in range(SUB):
                    copy(stg_cw.at[r, k], stg_cw.at[r, k],
                         snd_cw.at[s, k], rcv_cw.at[s, k], right).wait_recv()
                    if s < HCW - 1:
                        stg_cw[r, k] = (stg_cw[r, k]
                                        + partial_ref[r, k]).astype(
                                            jnp.bfloat16)
                        copy(stg_cw.at[r, k], stg_cw.at[r, k],
                             snd_cw.at[s + 1, k], rcv_cw.at[s + 1, k],
                             right).start()
                if s < HCCW:
                    r2 = rid[-6 + s]
                    for k in range(SUB):
                        copy(stg_ccw.at[r2, k], stg_ccw.at[r2, k],
                             snd_ccw.at[s, k], rcv_ccw.at[s, k],
                             left).wait_recv()
                        if s < HCCW - 1:
                            stg_ccw[r2, k] = (stg_ccw[r2, k]
                                              + partial_ref[r2, k]).astype(
                                                  jnp.bfloat16)
                            copy(stg_ccw.at[r2, k], stg_ccw.at[r2, k],
                                 snd_ccw.at[s + 1, k], rcv_ccw.at[s + 1, k],
                                 left).start()
            for k in range(SUB):
                act_ref[me, k] = (stg_cw[me, k] + stg_ccw[me, k]
                                  + partial_ref[me, k]).astype(jnp.bfloat16)
            drain_sends()

        def mk_compute(l):
            def compute_chunk(c):
                for k in range(SUB):
                    h = jnp.dot(act_ref[c, k], win_bf[l],
                                preferred_element_type=jnp.float32)
                    h = jnp.maximum(h, 0.0)
                    partial_ref[c, k] = jnp.dot(
                        h.astype(jnp.bfloat16), wout_bf[l],
                        preferred_element_type=jnp.float32)
            return compute_chunk

        for l, (wi, wo) in enumerate(((win0, wout0), (win1, wout1),
                                      (win2, wout2))):
            win_bf[l] = wi[...].astype(jnp.bfloat16)
            wout_bf[l] = wo[...].astype(jnp.bfloat16)

        barrier()
        for k in range(SUB):
            act_ref[me, k] = x_ref[pl.ds(k * Bs, Bs), :].astype(jnp.bfloat16)
        ring_allgather(compute_chunk=mk_compute(0))
        for l in range(3):
            barrier()
            ring_reducescatter()
            barrier()
            ring_allgather(compute_chunk=mk_compute(l + 1) if l < 2 else None)
        out_ref[...] = act_ref[...].astype(jnp.float32)

    out = pl.pallas_call(
        body,
        out_shape=jax.ShapeDtypeStruct((P, SUB, Bs, D), jnp.float32),
        in_specs=[pl.BlockSpec(memory_space=pltpu.VMEM)] * 7,
        out_specs=pl.BlockSpec(memory_space=pltpu.VMEM),
        scratch_shapes=[
            pltpu.VMEM((P, SUB, Bs, D), jnp.bfloat16),
            pltpu.VMEM((P, SUB, Bs, D), jnp.float32),
            pltpu.VMEM((P, SUB, Bs, D), jnp.bfloat16),
            pltpu.VMEM((P, SUB, Bs, D), jnp.bfloat16),
            pltpu.VMEM((3, D, H), jnp.bfloat16),
            pltpu.VMEM((3, H, D), jnp.bfloat16),
            pltpu.SemaphoreType.DMA((HCW, SUB)),
            pltpu.SemaphoreType.DMA((HCCW, SUB)),
            pltpu.SemaphoreType.DMA((HCW, SUB)),
            pltpu.SemaphoreType.DMA((HCCW, SUB)),
        ],
        compiler_params=pltpu.CompilerParams(collective_id=0),
    )(x, Win0, Wout0, Win1, Wout1, Win2, Wout2)
    return out.reshape(P * B, D)
